# baseline (speedup 1.0000x reference)
import jax
import jax.numpy as jnp
from jax import lax
from jax.experimental import pallas as pl
from jax.experimental.pallas import tpu as pltpu
from jax.experimental.pallas import tpu_sc as plsc

VOCAB = 1000000
D = 64
BATCH = 4096
HIST = 200

_info = plsc.get_sparse_core_info()
NW = _info.num_cores * _info.num_subcores
BATCH_PER_W = BATCH // NW
STEPS = BATCH_PER_W
_GROUPS = [(g * 16, min(16, HIST - g * 16)) for g in range((HIST + 15) // 16)]


def _gather_body(table_hbm, idx_hbm, out_hbm,
                 idx0, idx1, rows0, rows1, gs0, gs1, ss0, ss1):
    wid = lax.axis_index("s") * _info.num_cores + lax.axis_index("c")
    base = wid * BATCH_PER_W
    idx = (idx0, idx1)
    rows = (rows0, rows1)
    gsem = (gs0, gs1)
    ssem = (ss0, ss1)

    def g_start(i, b):
        pltpu.sync_copy(
            idx_hbm.at[pl.ds((base + i) * HIST, HIST)],
            idx[b].at[pl.ds(0, HIST)])
        for r0, cnt in _GROUPS:
            vec = idx[b][pl.ds(r0, 16)]
            for k in range(cnt):
                pltpu.async_copy(
                    table_hbm.at[pl.ds(vec[k], 1)],
                    rows[b].at[pl.ds(r0 + k, 1)], gsem[b])

    def g_wait(b):
        pltpu.make_async_copy(
            table_hbm.at[pl.ds(0, HIST)], rows[b], gsem[b]).wait()

    def s_start(i, b):
        pltpu.async_copy(rows[b], out_hbm.at[base + i], ssem[b])

    def s_wait(b):
        pltpu.make_async_copy(rows[b], out_hbm.at[0], ssem[b]).wait()

    g_start(0, 0)
    g_wait(0)
    s_start(0, 0)
    g_start(1, 1)
    g_wait(1)
    s_start(1, 1)
    g_start(2, 0)

    def pair(k, _):
        for off in (0, 1):
            i = 2 + 2 * k + off
            b = off
            g_wait(b)
            s_wait(b)
            s_start(i, b)
            g_start(i + 1, 1 - b)
        return None

    lax.fori_loop(0, (STEPS - 4) // 2, pair, None)

    g_wait(0)
    s_wait(0)
    s_start(STEPS - 2, 0)
    g_start(STEPS - 1, 1)
    g_wait(1)
    s_wait(1)
    s_start(STEPS - 1, 1)
    s_wait(0)
    s_wait(1)


_gather_call = pl.kernel(
    _gather_body,
    mesh=plsc.VectorSubcoreMesh(core_axis_name="c", subcore_axis_name="s"),
    out_type=jax.ShapeDtypeStruct((BATCH, HIST, D), jnp.float32),
    scratch_types=[
        pltpu.VMEM((208,), jnp.int32),
        pltpu.VMEM((208,), jnp.int32),
        pltpu.VMEM((HIST, D), jnp.float32),
        pltpu.VMEM((HIST, D), jnp.float32),
        pltpu.SemaphoreType.DMA,
        pltpu.SemaphoreType.DMA,
        pltpu.SemaphoreType.DMA,
        pltpu.SemaphoreType.DMA,
    ],
    compiler_params=pltpu.CompilerParams(use_tc_tiling_on_sc=True),
)


def kernel(x, embeddings):
    idx = x.reshape(BATCH * HIST).astype(jnp.int32)
    return _gather_call(embeddings, idx)


# speedup vs baseline: 1.1747x; 1.1747x over previous
import jax
import jax.numpy as jnp
from jax import lax
from jax.experimental import pallas as pl
from jax.experimental.pallas import tpu as pltpu
from jax.experimental.pallas import tpu_sc as plsc

VOCAB = 1000000
D = 64
BATCH = 4096
HIST = 200
B = BATCH * HIST

_info = plsc.get_sparse_core_info()
NW = _info.num_cores * _info.num_subcores
B_PER_W = B // NW
CHUNK = 256
STEPS = B_PER_W // CHUNK


def _gather_body(table_hbm, idx_hbm, out_hbm,
                 idx_v, rows0, rows1, gs0, gs1, ss0, ss1):
    wid = lax.axis_index("s") * _info.num_cores + lax.axis_index("c")
    base = wid * B_PER_W
    rows = (rows0, rows1)
    gsem = (gs0, gs1)
    ssem = (ss0, ss1)

    def g_start(i, b):
        pltpu.sync_copy(idx_hbm.at[pl.ds(base + i * CHUNK, CHUNK)], idx_v)

        def group(g, _):
            r0 = g * 16
            vec = idx_v[pl.ds(r0, 16)]
            for k in range(16):
                pltpu.async_copy(
                    table_hbm.at[pl.ds(vec[k], 1), pl.ds(0, D)],
                    rows[b].at[pl.ds(r0 + k, 1), pl.ds(0, D)], gsem[b])
            return None

        lax.fori_loop(0, CHUNK // 16, group, None)

    def g_wait(b):
        pltpu.make_async_copy(
            table_hbm.at[pl.ds(0, CHUNK)], rows[b], gsem[b]).wait()

    def s_start(i, b):
        pltpu.async_copy(
            rows[b], out_hbm.at[pl.ds(base + i * CHUNK, CHUNK)], ssem[b])

    def s_wait(b):
        pltpu.make_async_copy(
            rows[b], out_hbm.at[pl.ds(base, CHUNK)], ssem[b]).wait()

    def uniform(i, b):
        nb = 1 - b
        g_wait(b)
        s_start(i, b)
        s_wait(nb)
        g_start(i + 1, nb)

    g_start(0, 0)
    g_wait(0)
    s_start(0, 0)
    g_start(1, 1)

    def pair(k, _):
        uniform(2 * k + 1, 1)
        uniform(2 * k + 2, 0)
        return None

    lax.fori_loop(0, (STEPS - 2) // 2, pair, None)

    g_wait(1)
    s_start(STEPS - 1, 1)
    s_wait(0)
    s_wait(1)


_gather_call = pl.kernel(
    _gather_body,
    mesh=plsc.VectorSubcoreMesh(core_axis_name="c", subcore_axis_name="s"),
    out_type=jax.ShapeDtypeStruct((B, D), jnp.float32),
    scratch_types=[
        pltpu.VMEM((CHUNK,), jnp.int32),
        pltpu.VMEM((CHUNK, D), jnp.float32),
        pltpu.VMEM((CHUNK, D), jnp.float32),
        pltpu.SemaphoreType.DMA,
        pltpu.SemaphoreType.DMA,
        pltpu.SemaphoreType.DMA,
        pltpu.SemaphoreType.DMA,
    ],
    compiler_params=pltpu.CompilerParams(
        use_tc_tiling_on_sc=True, needs_layout_passes=False),
)


def kernel(x, embeddings):
    idx = x.reshape(B).astype(jnp.int32)
    out = _gather_call(embeddings, idx)
    return out.reshape(BATCH, HIST, D)


# speedup vs baseline: 1.2186x; 1.0373x over previous
import jax
import jax.numpy as jnp
from jax import lax
from jax.experimental import pallas as pl
from jax.experimental.pallas import tpu as pltpu
from jax.experimental.pallas import tpu_sc as plsc

VOCAB = 1000000
D = 64
BATCH = 4096
HIST = 200
B = BATCH * HIST

_info = plsc.get_sparse_core_info()
NW = _info.num_cores * _info.num_subcores
B_PER_W = B // NW
CHUNK = 400
STEPS = B_PER_W // CHUNK


def _gather_body(table_hbm, idx_hbm, out_hbm,
                 idx_v, rows0, rows1, gs0, gs1, ss0, ss1):
    wid = lax.axis_index("s") * _info.num_cores + lax.axis_index("c")
    base = wid * B_PER_W
    rows = (rows0, rows1)
    gsem = (gs0, gs1)
    ssem = (ss0, ss1)

    def g_start(i, b):
        pltpu.sync_copy(idx_hbm.at[pl.ds(base + i * CHUNK, CHUNK)], idx_v)

        def group(g, _):
            r0 = g * 16
            vec = idx_v[pl.ds(r0, 16)]
            for k in range(16):
                pltpu.async_copy(
                    table_hbm.at[pl.ds(vec[k], 1), pl.ds(0, D)],
                    rows[b].at[pl.ds(r0 + k, 1), pl.ds(0, D)], gsem[b])
            return None

        lax.fori_loop(0, CHUNK // 16, group, None)

    def g_wait(b):
        pltpu.make_async_copy(
            table_hbm.at[pl.ds(0, CHUNK)], rows[b], gsem[b]).wait()

    def s_start(i, b):
        pltpu.async_copy(
            rows[b], out_hbm.at[pl.ds(base + i * CHUNK, CHUNK)], ssem[b])

    def s_wait(b):
        pltpu.make_async_copy(
            rows[b], out_hbm.at[pl.ds(base, CHUNK)], ssem[b]).wait()

    def uniform(i, b):
        nb = 1 - b
        g_wait(b)
        s_start(i, b)
        s_wait(nb)
        g_start(i + 1, nb)

    g_start(0, 0)
    g_wait(0)
    s_start(0, 0)
    g_start(1, 1)

    def pair(k, _):
        uniform(2 * k + 1, 1)
        uniform(2 * k + 2, 0)
        return None

    lax.fori_loop(0, (STEPS - 2) // 2, pair, None)

    g_wait(1)
    s_start(STEPS - 1, 1)
    s_wait(0)
    s_wait(1)


_gather_call = pl.kernel(
    _gather_body,
    mesh=plsc.VectorSubcoreMesh(core_axis_name="c", subcore_axis_name="s"),
    out_type=jax.ShapeDtypeStruct((B, D), jnp.float32),
    scratch_types=[
        pltpu.VMEM((CHUNK,), jnp.int32),
        pltpu.VMEM((CHUNK, D), jnp.float32),
        pltpu.VMEM((CHUNK, D), jnp.float32),
        pltpu.SemaphoreType.DMA,
        pltpu.SemaphoreType.DMA,
        pltpu.SemaphoreType.DMA,
        pltpu.SemaphoreType.DMA,
    ],
    compiler_params=pltpu.CompilerParams(use_tc_tiling_on_sc=True),
)


def kernel(x, embeddings):
    idx = x.reshape(B).astype(jnp.int32)
    out = _gather_call(embeddings, idx)
    return out.reshape(BATCH, HIST, D)
